# baseline (speedup 1.0000x reference)
import jax
import jax.numpy as jnp
from jax import lax
from jax.experimental import pallas as pl
from jax.experimental.pallas import tpu as pltpu

_CH = 128


def _body(feat_ref, anc_ref, conf_ref, conf_t_ref, cfeat_ref, canc_ref,
          mask_ref, out_feat_ref, out_anc_ref, cmax_scr):
    n = feat_ref.shape[1]
    nc = cfeat_ref.shape[1]
    k = n - nc
    npad = cmax_scr.shape[0]

    conf = conf_ref[0]
    conf_t = conf_t_ref[0]
    cmax_col = jnp.max(conf, axis=1, keepdims=True)
    cmax_row = jnp.max(conf_t, axis=0, keepdims=True)
    pad = jnp.full((npad - n, 1), -jnp.inf, jnp.float32)
    cmax_scr[...] = jnp.concatenate([cmax_col, pad], axis=0)

    def rank_step(i, acc):
        j0 = pl.multiple_of(i * _CH, _CH)
        cj = cmax_scr[pl.ds(j0, _CH), :]
        jj = lax.broadcasted_iota(jnp.int32, (_CH, n), 0) + i * _CH
        ii = lax.broadcasted_iota(jnp.int32, (_CH, n), 1)
        beats = (cj > cmax_row) | ((cj == cmax_row) & (jj < ii))
        return acc + jnp.sum(beats.astype(jnp.int32), axis=0, keepdims=True)

    rank = lax.fori_loop(0, npad // _CH, rank_step,
                         jnp.zeros((1, n), jnp.int32))

    r_iota = lax.broadcasted_iota(jnp.int32, (k, n), 0)
    w = (rank == r_iota).astype(jnp.bfloat16)

    feat = feat_ref[0]
    anc = anc_ref[0]
    sel_feat = jnp.dot(w, feat.astype(jnp.bfloat16),
                       preferred_element_type=jnp.float32)
    sel_anc = jnp.dot(w, anc.astype(jnp.bfloat16),
                      preferred_element_type=jnp.float32)

    m = mask_ref[pl.program_id(0)] != 0
    out_feat_ref[0] = jnp.concatenate(
        [jnp.where(m, cfeat_ref[0], feat[:nc]),
         jnp.where(m, sel_feat, feat[nc:])], axis=0)
    out_anc_ref[0] = jnp.concatenate(
        [jnp.where(m, canc_ref[0], anc[:nc]),
         jnp.where(m, sel_anc, anc[nc:])], axis=0)


def kernel(instance_feature, anchor, confidence, cached_feature,
           cached_anchor, mask, interpret=False):
    bs, n, d = instance_feature.shape
    a = anchor.shape[2]
    c = confidence.shape[2]
    nc = cached_feature.shape[1]
    mask_i32 = mask.astype(jnp.int32)
    conf_t = jnp.transpose(confidence, (0, 2, 1))
    npad = ((n + _CH - 1) // _CH) * _CH

    out_feat, out_anc = pl.pallas_call(
        _body,
        grid=(bs,),
        in_specs=[
            pl.BlockSpec((1, n, d), lambda b: (b, 0, 0)),
            pl.BlockSpec((1, n, a), lambda b: (b, 0, 0)),
            pl.BlockSpec((1, n, c), lambda b: (b, 0, 0)),
            pl.BlockSpec((1, c, n), lambda b: (b, 0, 0)),
            pl.BlockSpec((1, nc, d), lambda b: (b, 0, 0)),
            pl.BlockSpec((1, nc, a), lambda b: (b, 0, 0)),
            pl.BlockSpec(memory_space=pltpu.SMEM),
        ],
        out_specs=[
            pl.BlockSpec((1, n, d), lambda b: (b, 0, 0)),
            pl.BlockSpec((1, n, a), lambda b: (b, 0, 0)),
        ],
        out_shape=[
            jax.ShapeDtypeStruct((bs, n, d), jnp.float32),
            jax.ShapeDtypeStruct((bs, n, a), jnp.float32),
        ],
        scratch_shapes=[pltpu.VMEM((npad, 1), jnp.float32)],
        interpret=interpret,
    )(instance_feature, anchor, confidence, conf_t, cached_feature,
      cached_anchor, mask_i32)
    return out_feat, out_anc
